# baseline (speedup 1.0000x reference)
import functools

import jax
import jax.numpy as jnp
from jax import lax
from jax.experimental import pallas as pl
from jax.experimental.pallas import tpu as pltpu
from jax.experimental.pallas import tpu_sc as plsc

N_ROWS = 320000
D = 128
NUM_GRAPHS = 512
LANES = 16
N_SUBCORES = 16
ACC = NUM_GRAPHS * LANES
G_PER_TILE = NUM_GRAPHS // N_SUBCORES
_R = 64

_VIEW_ROWS = 2500
_HALF_BLOCKS = 20


def _energy_body(x_ref, o_ref):
    x = x_ref[...]
    o_ref[...] = 0.5 * jnp.sum(x * x, axis=-1)


def _make_energy(block_off, out_rows):
    return pl.pallas_call(
        _energy_body,
        grid=(_HALF_BLOCKS,),
        in_specs=[pl.BlockSpec((_R, D, D), lambda i: (i + block_off, 0, 0))],
        out_specs=pl.BlockSpec((_R, D), lambda i: (i, 0)),
        out_shape=jax.ShapeDtypeStruct((out_rows, D), jnp.float32),
        compiler_params=pltpu.CompilerParams(
            dimension_semantics=("arbitrary",),
        ),
    )


_energy0 = _make_energy(0, _R * _HALF_BLOCKS)
_energy1 = _make_energy(_HALF_BLOCKS, 2500 - _R * _HALF_BLOCKS)

_mesh = plsc.VectorSubcoreMesh(
    core_axis_name="c", subcore_axis_name="s", num_cores=1
)


def _make_scatter(koff, chunk, unroll, has_prev):
    scratch = [
        pltpu.VMEM((chunk,), jnp.float32),
        pltpu.VMEM((chunk,), jnp.int32),
        pltpu.VMEM((ACC,), jnp.float32),
        pltpu.VMEM((G_PER_TILE * LANES,), jnp.float32),
        pltpu.VMEM((N_SUBCORES, G_PER_TILE * LANES), jnp.float32),
        pltpu.VMEM((G_PER_TILE,), jnp.float32),
        pltpu.VMEM((G_PER_TILE,), jnp.float32),
        pltpu.VMEM_SHARED((N_SUBCORES, ACC), jnp.float32),
    ]

    @functools.partial(
        pl.kernel,
        mesh=_mesh,
        out_type=jax.ShapeDtypeStruct((NUM_GRAPHS,), jnp.float32),
        scratch_types=scratch,
        compiler_params=pltpu.CompilerParams(needs_layout_passes=False),
    )
    def _scatter(e_hbm, b_hbm, *rest):
        if has_prev:
            (prev_hbm, out_hbm, e_v, b_v, acc_v, sum_v,
             stage_v, res_v, prev_v, shared) = rest
        else:
            (out_hbm, e_v, b_v, acc_v, sum_v,
             stage_v, res_v, prev_v, shared) = rest
        sid = lax.axis_index("s")
        base = koff + sid * chunk

        pltpu.sync_copy(e_hbm.at[pl.ds(sid * chunk, chunk)], e_v)
        pltpu.sync_copy(b_hbm.at[pl.ds(base, chunk)], b_v)
        if has_prev:
            pltpu.sync_copy(prev_hbm.at[pl.ds(sid * G_PER_TILE, G_PER_TILE)],
                            prev_v)

        zeros16 = jnp.zeros((LANES,), jnp.float32)

        @plsc.parallel_loop(0, ACC // LANES, unroll=8)
        def _zero(i):
            acc_v[pl.ds(i * LANES, LANES)] = zeros16

        lane = lax.iota(jnp.int32, LANES)

        @plsc.parallel_loop(0, chunk // LANES, unroll=unroll)
        def _accum(i):
            s = pl.ds(i * LANES, LANES)
            idx = b_v[s]
            ev = e_v[s]
            plsc.addupdate_scatter(acc_v, [idx * LANES + lane], ev)

        pltpu.sync_copy(acc_v, shared.at[sid])
        plsc.subcore_barrier()

        goff = sid * G_PER_TILE * LANES
        pltpu.sync_copy(
            shared.at[:, pl.ds(goff, G_PER_TILE * LANES)], stage_v
        )

        @plsc.parallel_loop(0, G_PER_TILE, unroll=4)
        def _fold_tiles(c):
            s = pl.ds(c * LANES, LANES)
            tot = stage_v[0, s]
            for t in range(1, N_SUBCORES):
                tot = tot + stage_v[t, s]
            sum_v[s] = tot

        for c in range(G_PER_TILE // LANES):
            addr = c * LANES * LANES + lane * LANES
            tot = plsc.load_gather(sum_v, [addr])
            for l in range(1, LANES):
                tot = tot + plsc.load_gather(sum_v, [addr + l])
            if has_prev:
                tot = tot + prev_v[pl.ds(c * LANES, LANES)]
            res_v[pl.ds(c * LANES, LANES)] = tot

        pltpu.sync_copy(res_v, out_hbm.at[pl.ds(sid * G_PER_TILE,
                                                G_PER_TILE)])

    return _scatter


_E0 = _R * _HALF_BLOCKS * D
_scatter0 = _make_scatter(0, _E0 // N_SUBCORES, 8, False)
_scatter1 = _make_scatter(_E0, (N_ROWS - _E0) // N_SUBCORES, 5, True)


def kernel(X, batch, num_graphs):
    del num_graphs
    Xv = X.reshape(_VIEW_ROWS, D, D)
    b = batch.astype(jnp.int32)
    e0 = _energy0(Xv).reshape(-1)
    e1 = _energy1(Xv).reshape(-1)
    p0 = _scatter0(e0, b)
    return _scatter1(e1, b, p0)
